# Initial kernel scaffold; baseline (speedup 1.0000x reference)
import functools

import jax
import jax.numpy as jnp
from jax import lax
from jax.experimental import pallas as pl
from jax.experimental.pallas import tpu as pltpu
from jax.experimental.pallas import tpu_sc as plsc

N = 10000
E = 320000
IN_DIM = 128
HID = 128
OUT = 64
EPS = 0.3

_NC = 2
_NS = 16
_NW = _NC * _NS
_L = 16
_C = 128
_T = 10112
_EP = _NW * _T
_NCHUNK = _T // _C
_NP = 10240
_RPT = _NP // _NS

_R = 2048
_NBLK = _NP // _R

_HIGH = lax.Precision.HIGHEST


def _sc_mesh():
    return plsc.VectorSubcoreMesh(
        core_axis_name="c", subcore_axis_name="s",
        num_cores=_NC, num_subcores=_NS)


@functools.partial(
    pl.kernel,
    out_type=jax.ShapeDtypeStruct((_NC, _NP), jnp.float32),
    mesh=_sc_mesh(),
    scratch_types=[
        pltpu.VMEM((_C,), jnp.int32),
        pltpu.VMEM((_C,), jnp.float32),
        pltpu.VMEM_SHARED((_NP,), jnp.float32),
    ],
)
def _deg_kernel(dst_hbm, ones_hbm, zer_hbm, out_hbm, idx_v, ones_v, deg_sh):
    cid = lax.axis_index("c")
    sid = lax.axis_index("s")
    base = (sid * _NC + cid) * _T
    pltpu.sync_copy(ones_hbm, ones_v)

    @pl.when(sid == 0)
    def _():
        pltpu.sync_copy(zer_hbm, deg_sh)

    plsc.subcore_barrier()

    def body(k, carry):
        pltpu.sync_copy(dst_hbm.at[pl.ds(base + k * _C, _C)], idx_v)
        pltpu.sync_copy(ones_v, deg_sh.at[idx_v], add=True)
        return carry

    lax.fori_loop(0, _NCHUNK, body, 0)
    plsc.subcore_barrier()
    pltpu.sync_copy(deg_sh.at[pl.ds(sid * _RPT, _RPT)],
                    out_hbm.at[cid, pl.ds(sid * _RPT, _RPT)])


@functools.partial(
    pl.kernel,
    out_type=jax.ShapeDtypeStruct((_NC, _NP, HID), jnp.float32),
    mesh=_sc_mesh(),
    scratch_types=[
        pltpu.VMEM((_NP,), jnp.float32),
        pltpu.VMEM((_NP,), jnp.float32),
        pltpu.VMEM((_NP,), jnp.float32),
        pltpu.VMEM((_C,), jnp.int32),
        pltpu.VMEM((_C,), jnp.int32),
        pltpu.VMEM((_C,), jnp.float32),
        pltpu.VMEM((_C, HID), jnp.float32),
        pltpu.VMEM_SHARED((_NP, HID), jnp.float32),
        pltpu.SemaphoreType.DMA,
    ],
)
def _edge_kernel(src_hbm, dst_hbm, scal_hbm, x_hbm, zer_hbm, out_hbm,
                 adb_v, as_v, d_v, si_v, di_v, w_v, rows_v, z_sh, sem):
    cid = lax.axis_index("c")
    sid = lax.axis_index("s")
    base = (sid * _NC + cid) * _T
    pltpu.sync_copy(scal_hbm.at[0], adb_v)
    pltpu.sync_copy(scal_hbm.at[1], as_v)
    pltpu.sync_copy(scal_hbm.at[2], d_v)
    pltpu.sync_copy(zer_hbm.at[pl.ds(sid * _RPT, _RPT)],
                    z_sh.at[pl.ds(sid * _RPT, _RPT)])
    plsc.subcore_barrier()

    def chunk(k, carry):
        off = base + k * _C
        pltpu.sync_copy(src_hbm.at[pl.ds(off, _C)], si_v)
        pltpu.sync_copy(dst_hbm.at[pl.ds(off, _C)], di_v)
        pltpu.async_copy(x_hbm.at[si_v], rows_v, sem).wait()
        for j in range(_C // _L):
            s16 = si_v[pl.ds(j * _L, _L)]
            t16 = di_v[pl.ds(j * _L, _L)]
            a = plsc.load_gather(adb_v, [t16])
            b = plsc.load_gather(as_v, [s16])
            ds_ = plsc.load_gather(d_v, [s16])
            u = a + b
            ex = jnp.exp(-2.0 * jnp.abs(u))
            th = (1.0 - ex) / (1.0 + ex)
            th = jnp.where(u < 0.0, -th, th)
            w_v[pl.ds(j * _L, _L)] = th * ds_

        def escale(e, c2):
            wsp = plsc.load_gather(w_v, [jnp.full((_L,), e, jnp.int32)])
            for j in range(HID // _L):
                rows_v[e, pl.ds(j * _L, _L)] = rows_v[e, pl.ds(j * _L, _L)] * wsp
            return c2

        lax.fori_loop(0, _C, escale, 0)
        pltpu.sync_copy(rows_v, z_sh.at[di_v], add=True)
        return carry

    lax.fori_loop(0, _NCHUNK, chunk, 0)
    plsc.subcore_barrier()
    pltpu.sync_copy(z_sh.at[pl.ds(sid * _RPT, _RPT)],
                    out_hbm.at[cid, pl.ds(sid * _RPT, _RPT)])


def _dense1_body(h_ref, w1_ref, b1_ref, wgd_ref, wgs_ref, bg_ref, deg_ref,
                 x_ref, scal_ref):
    x = lax.dot_general(h_ref[...], w1_ref[...], (((1,), (1,)), ((), ())),
                        precision=_HIGH)
    x = jnp.maximum(x + b1_ref[...], 0.0)
    x_ref[...] = x
    adb = lax.dot_general(wgd_ref[...], x, (((1,), (1,)), ((), ())),
                          precision=_HIGH) + bg_ref[0, 0]
    asr = lax.dot_general(wgs_ref[...], x, (((1,), (1,)), ((), ())),
                          precision=_HIGH)
    d = lax.rsqrt(jnp.maximum(deg_ref[0:1, :] + deg_ref[1:2, :], 1.0))
    scal_ref[0:1, :] = adb
    scal_ref[1:2, :] = asr
    scal_ref[2:3, :] = d
    scal_ref[3:8, :] = jnp.zeros((5, _R), jnp.float32)


def _dense1(h_p, W1, b1r, wgd, wgs, bgb, deg):
    return pl.pallas_call(
        _dense1_body,
        grid=(_NBLK,),
        in_specs=[
            pl.BlockSpec((_R, IN_DIM), lambda i: (i, 0)),
            pl.BlockSpec((HID, IN_DIM), lambda i: (0, 0)),
            pl.BlockSpec((1, HID), lambda i: (0, 0)),
            pl.BlockSpec((1, HID), lambda i: (0, 0)),
            pl.BlockSpec((1, HID), lambda i: (0, 0)),
            pl.BlockSpec((1, HID), lambda i: (0, 0)),
            pl.BlockSpec((2, _R), lambda i: (0, i)),
        ],
        out_specs=[
            pl.BlockSpec((_R, HID), lambda i: (i, 0)),
            pl.BlockSpec((8, _R), lambda i: (0, i)),
        ],
        out_shape=[
            jax.ShapeDtypeStruct((_NP, HID), jnp.float32),
            jax.ShapeDtypeStruct((8, _NP), jnp.float32),
        ],
    )(h_p, W1, b1r, wgd, wgs, bgb, deg)


def _dense2_body(x0_ref, za_ref, zb_ref, deg_ref, degc_ref, wgd_ref, wgs_ref,
                 bg_ref, x1_ref, scal_ref):
    dc = lax.rsqrt(jnp.maximum(degc_ref[:, 0:1] + degc_ref[:, 1:2], 1.0))
    x1 = EPS * x0_ref[...] + dc * (za_ref[...] + zb_ref[...])
    x1_ref[...] = x1
    adb = lax.dot_general(wgd_ref[...], x1, (((1,), (1,)), ((), ())),
                          precision=_HIGH) + bg_ref[0, 0]
    asr = lax.dot_general(wgs_ref[...], x1, (((1,), (1,)), ((), ())),
                          precision=_HIGH)
    d = lax.rsqrt(jnp.maximum(deg_ref[0:1, :] + deg_ref[1:2, :], 1.0))
    scal_ref[0:1, :] = adb
    scal_ref[1:2, :] = asr
    scal_ref[2:3, :] = d
    scal_ref[3:8, :] = jnp.zeros((5, _R), jnp.float32)


def _dense2(x0, za, zb, deg, degc, wgd, wgs, bgb):
    return pl.pallas_call(
        _dense2_body,
        grid=(_NBLK,),
        in_specs=[
            pl.BlockSpec((_R, HID), lambda i: (i, 0)),
            pl.BlockSpec((_R, HID), lambda i: (i, 0)),
            pl.BlockSpec((_R, HID), lambda i: (i, 0)),
            pl.BlockSpec((2, _R), lambda i: (0, i)),
            pl.BlockSpec((_R, 2), lambda i: (i, 0)),
            pl.BlockSpec((1, HID), lambda i: (0, 0)),
            pl.BlockSpec((1, HID), lambda i: (0, 0)),
            pl.BlockSpec((1, HID), lambda i: (0, 0)),
        ],
        out_specs=[
            pl.BlockSpec((_R, HID), lambda i: (i, 0)),
            pl.BlockSpec((8, _R), lambda i: (0, i)),
        ],
        out_shape=[
            jax.ShapeDtypeStruct((_NP, HID), jnp.float32),
            jax.ShapeDtypeStruct((8, _NP), jnp.float32),
        ],
    )(x0, za, zb, deg, degc, wgd, wgs, bgb)


def _dense3_body(x0_ref, za_ref, zb_ref, degc_ref, w2_ref, b2_ref, o_ref):
    dc = lax.rsqrt(jnp.maximum(degc_ref[:, 0:1] + degc_ref[:, 1:2], 1.0))
    x2 = EPS * x0_ref[...] + dc * (za_ref[...] + zb_ref[...])
    o = lax.dot_general(x2, w2_ref[...], (((1,), (1,)), ((), ())),
                        precision=_HIGH) + b2_ref[...]
    m = jnp.max(o, axis=1, keepdims=True)
    s = o - m
    lse = jnp.log(jnp.sum(jnp.exp(s), axis=1, keepdims=True))
    o_ref[...] = s - lse


def _dense3(x0, za, zb, degc, W2, b2r):
    return pl.pallas_call(
        _dense3_body,
        grid=(_NBLK,),
        in_specs=[
            pl.BlockSpec((_R, HID), lambda i: (i, 0)),
            pl.BlockSpec((_R, HID), lambda i: (i, 0)),
            pl.BlockSpec((_R, HID), lambda i: (i, 0)),
            pl.BlockSpec((_R, 2), lambda i: (i, 0)),
            pl.BlockSpec((OUT, HID), lambda i: (0, 0)),
            pl.BlockSpec((1, OUT), lambda i: (0, 0)),
        ],
        out_specs=pl.BlockSpec((_R, OUT), lambda i: (i, 0)),
        out_shape=jax.ShapeDtypeStruct((_NP, OUT), jnp.float32),
    )(x0, za, zb, degc, W2, b2r)


def kernel(h, edge_index, W1, b1, Wg1, bg1, Wg2, bg2, W2, b2):
    src = edge_index[0].astype(jnp.int32)
    dst = edge_index[1].astype(jnp.int32)
    src_p = jnp.concatenate([src, jnp.zeros((_EP - E,), jnp.int32)])
    dst_p = jnp.concatenate([dst, jnp.full((_EP - E,), N, jnp.int32)])
    h_p = jnp.pad(h, ((0, _NP - N), (0, 0)))
    wg1d, wg1s = Wg1[:, :HID], Wg1[:, HID:]
    wg2d, wg2s = Wg2[:, :HID], Wg2[:, HID:]
    b1r = b1.reshape(1, HID)
    b2r = b2.reshape(1, OUT)
    bg1b = jnp.broadcast_to(bg1.reshape(1, 1), (1, HID))
    bg2b = jnp.broadcast_to(bg2.reshape(1, 1), (1, HID))
    ones_c = jnp.ones((_C,), jnp.float32)
    zeros_node = jnp.zeros((_NP,), jnp.float32)
    zeros_rows = jnp.zeros((_NP, HID), jnp.float32)

    deg = _deg_kernel(dst_p, ones_c, zeros_node)
    degc = deg.T
    x0, scal1 = _dense1(h_p, W1, b1r, wg1d, wg1s, bg1b, deg)
    z1 = _edge_kernel(src_p, dst_p, scal1, x0, zeros_rows)
    x1, scal2 = _dense2(x0, z1[0], z1[1], deg, degc, wg2d, wg2s, bg2b)
    z2 = _edge_kernel(src_p, dst_p, scal2, x1, zeros_rows)
    out = _dense3(x0, z2[0], z2[1], degc, W2, b2r)
    return out[:N]


# speedup vs baseline: 8.4708x; 8.4708x over previous
import functools

import jax
import jax.numpy as jnp
from jax import lax
from jax.experimental import pallas as pl
from jax.experimental.pallas import tpu as pltpu
from jax.experimental.pallas import tpu_sc as plsc

N = 10000
E = 320000
IN_DIM = 128
HID = 128
OUT = 64
EPS = 0.3

_NC = 2
_NS = 16
_NW = _NC * _NS
_L = 16
_C = 128
_T = 10112
_EP = _NW * _T
_NCHUNK = _T // _C
_NP = 10240
_RPT = _NP // _NS

_R = 2048
_NBLK = _NP // _R

_HIGH = lax.Precision.HIGHEST


def _sc_mesh():
    return plsc.VectorSubcoreMesh(
        core_axis_name="c", subcore_axis_name="s",
        num_cores=_NC, num_subcores=_NS)


@functools.partial(
    pl.kernel,
    out_type=jax.ShapeDtypeStruct((_NC, _NP), jnp.float32),
    mesh=_sc_mesh(),
    compiler_params=pltpu.CompilerParams(needs_layout_passes=False),
    scratch_types=[
        pltpu.VMEM((_C,), jnp.int32),
        pltpu.VMEM((_C,), jnp.float32),
        pltpu.VMEM_SHARED((_NP,), jnp.float32),
    ],
)
def _deg_kernel(dst_hbm, ones_hbm, zer_hbm, out_hbm, idx_v, ones_v, deg_sh):
    cid = lax.axis_index("c")
    sid = lax.axis_index("s")
    base = (sid * _NC + cid) * _T
    pltpu.sync_copy(ones_hbm, ones_v)

    @pl.when(sid == 0)
    def _():
        pltpu.sync_copy(zer_hbm, deg_sh)

    plsc.subcore_barrier()

    def body(k, carry):
        pltpu.sync_copy(dst_hbm.at[pl.ds(base + k * _C, _C)], idx_v)
        pltpu.sync_copy(ones_v, deg_sh.at[idx_v], add=True)
        return carry

    lax.fori_loop(0, _NCHUNK, body, 0)
    plsc.subcore_barrier()
    pltpu.sync_copy(deg_sh.at[pl.ds(sid * _RPT, _RPT)],
                    out_hbm.at[cid, pl.ds(sid * _RPT, _RPT)])


@functools.partial(
    pl.kernel,
    out_type=jax.ShapeDtypeStruct((_NC, _NP, HID), jnp.float32),
    mesh=_sc_mesh(),
    compiler_params=pltpu.CompilerParams(needs_layout_passes=False),
    scratch_types=[
        pltpu.VMEM((_NP,), jnp.float32),
        pltpu.VMEM((_NP,), jnp.float32),
        pltpu.VMEM((_NP,), jnp.float32),
        pltpu.VMEM((_C,), jnp.int32),
        pltpu.VMEM((_C,), jnp.int32),
        pltpu.VMEM((_C,), jnp.float32),
        pltpu.VMEM((_C, HID), jnp.float32),
        pltpu.VMEM_SHARED((_NP, HID), jnp.float32),
        pltpu.SemaphoreType.DMA,
    ],
)
def _edge_kernel(src_hbm, dst_hbm, scal_hbm, x_hbm, zer_hbm, out_hbm,
                 adb_v, as_v, d_v, si_v, di_v, w_v, rows_v, z_sh, sem):
    cid = lax.axis_index("c")
    sid = lax.axis_index("s")
    base = (sid * _NC + cid) * _T
    pltpu.sync_copy(scal_hbm.at[0], adb_v)
    pltpu.sync_copy(scal_hbm.at[1], as_v)
    pltpu.sync_copy(scal_hbm.at[2], d_v)
    pltpu.sync_copy(zer_hbm.at[pl.ds(sid * _RPT, _RPT)],
                    z_sh.at[pl.ds(sid * _RPT, _RPT)])
    plsc.subcore_barrier()

    def chunk(k, carry):
        off = base + k * _C
        pltpu.sync_copy(src_hbm.at[pl.ds(off, _C)], si_v)
        pltpu.sync_copy(dst_hbm.at[pl.ds(off, _C)], di_v)
        pltpu.async_copy(x_hbm.at[si_v], rows_v, sem).wait()
        for j in range(_C // _L):
            s16 = si_v[pl.ds(j * _L, _L)]
            t16 = di_v[pl.ds(j * _L, _L)]
            a = plsc.load_gather(adb_v, [t16])
            b = plsc.load_gather(as_v, [s16])
            ds_ = plsc.load_gather(d_v, [s16])
            u = a + b
            ex = jnp.exp(-2.0 * jnp.abs(u))
            th = (1.0 - ex) / (1.0 + ex)
            th = jnp.where(u < 0.0, -th, th)
            w_v[pl.ds(j * _L, _L)] = th * ds_

        def escale(e, c2):
            wsp = plsc.load_gather(w_v, [jnp.full((_L,), e, jnp.int32)])
            for j in range(HID // _L):
                rows_v[e, pl.ds(j * _L, _L)] = rows_v[e, pl.ds(j * _L, _L)] * wsp
            return c2

        lax.fori_loop(0, _C, escale, 0)
        pltpu.sync_copy(rows_v, z_sh.at[di_v], add=True)
        return carry

    lax.fori_loop(0, _NCHUNK, chunk, 0)
    plsc.subcore_barrier()
    pltpu.sync_copy(z_sh.at[pl.ds(sid * _RPT, _RPT)],
                    out_hbm.at[cid, pl.ds(sid * _RPT, _RPT)])


def _dense1_body(h_ref, w1_ref, b1_ref, wgd_ref, wgs_ref, bg_ref, deg_ref,
                 x_ref, scal_ref):
    x = lax.dot_general(h_ref[...], w1_ref[...], (((1,), (1,)), ((), ())),
                        precision=_HIGH)
    x = jnp.maximum(x + b1_ref[...], 0.0)
    x_ref[...] = x
    adb = lax.dot_general(wgd_ref[...], x, (((1,), (1,)), ((), ())),
                          precision=_HIGH) + bg_ref[0, 0]
    asr = lax.dot_general(wgs_ref[...], x, (((1,), (1,)), ((), ())),
                          precision=_HIGH)
    d = lax.rsqrt(jnp.maximum(deg_ref[0:1, :] + deg_ref[1:2, :], 1.0))
    scal_ref[0:1, :] = adb
    scal_ref[1:2, :] = asr
    scal_ref[2:3, :] = d
    scal_ref[3:8, :] = jnp.zeros((5, _R), jnp.float32)


def _dense1(h_p, W1, b1r, wgd, wgs, bgb, deg):
    return pl.pallas_call(
        _dense1_body,
        grid=(_NBLK,),
        in_specs=[
            pl.BlockSpec((_R, IN_DIM), lambda i: (i, 0)),
            pl.BlockSpec((HID, IN_DIM), lambda i: (0, 0)),
            pl.BlockSpec((1, HID), lambda i: (0, 0)),
            pl.BlockSpec((1, HID), lambda i: (0, 0)),
            pl.BlockSpec((1, HID), lambda i: (0, 0)),
            pl.BlockSpec((1, HID), lambda i: (0, 0)),
            pl.BlockSpec((2, _R), lambda i: (0, i)),
        ],
        out_specs=[
            pl.BlockSpec((_R, HID), lambda i: (i, 0)),
            pl.BlockSpec((8, _R), lambda i: (0, i)),
        ],
        out_shape=[
            jax.ShapeDtypeStruct((_NP, HID), jnp.float32),
            jax.ShapeDtypeStruct((8, _NP), jnp.float32),
        ],
    )(h_p, W1, b1r, wgd, wgs, bgb, deg)


def _dense2_body(x0_ref, za_ref, zb_ref, deg_ref, degc_ref, wgd_ref, wgs_ref,
                 bg_ref, x1_ref, scal_ref):
    dc = lax.rsqrt(jnp.maximum(degc_ref[:, 0:1] + degc_ref[:, 1:2], 1.0))
    x1 = EPS * x0_ref[...] + dc * (za_ref[...] + zb_ref[...])
    x1_ref[...] = x1
    adb = lax.dot_general(wgd_ref[...], x1, (((1,), (1,)), ((), ())),
                          precision=_HIGH) + bg_ref[0, 0]
    asr = lax.dot_general(wgs_ref[...], x1, (((1,), (1,)), ((), ())),
                          precision=_HIGH)
    d = lax.rsqrt(jnp.maximum(deg_ref[0:1, :] + deg_ref[1:2, :], 1.0))
    scal_ref[0:1, :] = adb
    scal_ref[1:2, :] = asr
    scal_ref[2:3, :] = d
    scal_ref[3:8, :] = jnp.zeros((5, _R), jnp.float32)


def _dense2(x0, za, zb, deg, degc, wgd, wgs, bgb):
    return pl.pallas_call(
        _dense2_body,
        grid=(_NBLK,),
        in_specs=[
            pl.BlockSpec((_R, HID), lambda i: (i, 0)),
            pl.BlockSpec((_R, HID), lambda i: (i, 0)),
            pl.BlockSpec((_R, HID), lambda i: (i, 0)),
            pl.BlockSpec((2, _R), lambda i: (0, i)),
            pl.BlockSpec((_R, 2), lambda i: (i, 0)),
            pl.BlockSpec((1, HID), lambda i: (0, 0)),
            pl.BlockSpec((1, HID), lambda i: (0, 0)),
            pl.BlockSpec((1, HID), lambda i: (0, 0)),
        ],
        out_specs=[
            pl.BlockSpec((_R, HID), lambda i: (i, 0)),
            pl.BlockSpec((8, _R), lambda i: (0, i)),
        ],
        out_shape=[
            jax.ShapeDtypeStruct((_NP, HID), jnp.float32),
            jax.ShapeDtypeStruct((8, _NP), jnp.float32),
        ],
    )(x0, za, zb, deg, degc, wgd, wgs, bgb)


def _dense3_body(x0_ref, za_ref, zb_ref, degc_ref, w2_ref, b2_ref, o_ref):
    dc = lax.rsqrt(jnp.maximum(degc_ref[:, 0:1] + degc_ref[:, 1:2], 1.0))
    x2 = EPS * x0_ref[...] + dc * (za_ref[...] + zb_ref[...])
    o = lax.dot_general(x2, w2_ref[...], (((1,), (1,)), ((), ())),
                        precision=_HIGH) + b2_ref[...]
    m = jnp.max(o, axis=1, keepdims=True)
    s = o - m
    lse = jnp.log(jnp.sum(jnp.exp(s), axis=1, keepdims=True))
    o_ref[...] = s - lse


def _dense3(x0, za, zb, degc, W2, b2r):
    return pl.pallas_call(
        _dense3_body,
        grid=(_NBLK,),
        in_specs=[
            pl.BlockSpec((_R, HID), lambda i: (i, 0)),
            pl.BlockSpec((_R, HID), lambda i: (i, 0)),
            pl.BlockSpec((_R, HID), lambda i: (i, 0)),
            pl.BlockSpec((_R, 2), lambda i: (i, 0)),
            pl.BlockSpec((OUT, HID), lambda i: (0, 0)),
            pl.BlockSpec((1, OUT), lambda i: (0, 0)),
        ],
        out_specs=pl.BlockSpec((_R, OUT), lambda i: (i, 0)),
        out_shape=jax.ShapeDtypeStruct((_NP, OUT), jnp.float32),
    )(x0, za, zb, degc, W2, b2r)


def kernel(h, edge_index, W1, b1, Wg1, bg1, Wg2, bg2, W2, b2):
    src = edge_index[0].astype(jnp.int32)
    dst = edge_index[1].astype(jnp.int32)
    src_p = jnp.concatenate([src, jnp.zeros((_EP - E,), jnp.int32)])
    dst_p = jnp.concatenate([dst, jnp.full((_EP - E,), N, jnp.int32)])
    h_p = jnp.pad(h, ((0, _NP - N), (0, 0)))
    wg1d, wg1s = Wg1[:, :HID], Wg1[:, HID:]
    wg2d, wg2s = Wg2[:, :HID], Wg2[:, HID:]
    b1r = b1.reshape(1, HID)
    b2r = b2.reshape(1, OUT)
    bg1b = jnp.broadcast_to(bg1.reshape(1, 1), (1, HID))
    bg2b = jnp.broadcast_to(bg2.reshape(1, 1), (1, HID))
    ones_c = jnp.ones((_C,), jnp.float32)
    zeros_node = jnp.zeros((_NP,), jnp.float32)
    zeros_rows = jnp.zeros((_NP, HID), jnp.float32)

    deg = _deg_kernel(dst_p, ones_c, zeros_node)
    degc = deg.T
    x0, scal1 = _dense1(h_p, W1, b1r, wg1d, wg1s, bg1b, deg)
    z1 = _edge_kernel(src_p, dst_p, scal1, x0, zeros_rows)
    x1, scal2 = _dense2(x0, z1[0], z1[1], deg, degc, wg2d, wg2s, bg2b)
    z2 = _edge_kernel(src_p, dst_p, scal2, x1, zeros_rows)
    out = _dense3(x0, z2[0], z2[1], degc, W2, b2r)
    return out[:N]
